# Initial kernel scaffold; baseline (speedup 1.0000x reference)
import jax
import jax.numpy as jnp
from jax.experimental import pallas as pl


def kernel(x, edge_index, batch, W_in, b_in, g_in, be_in, p_topk, W1, b1, g1, be1, W2, b2, g2, be2, W3, b3):
    raise NotImplementedError("write your pallas kernel here")


# speedup vs baseline: 5.7926x; 5.7926x over previous
import jax
import jax.numpy as jnp
from jax.experimental import pallas as pl

N = 10000
D = 256
H = 256
B = 50
OUT = 2
RATIO = 0.8
EPS = 1e-5
BLK = 2000
NSTEP = N // BLK

_KEY_NEG_OFF = -2147483648
_LO0 = -2139095040
_HI0 = 2139095041


def _a_body(x_ref, Win_ref, bin_ref, z_ref, s_ref):
    i = pl.program_id(0)
    dn = (((1,), (1,)), ((), ()))
    z = jax.lax.dot_general(x_ref[...], Win_ref[...], dn,
                            preferred_element_type=jnp.float32, precision=None) + bin_ref[...]
    z_ref[...] = z

    @pl.when(i == 0)
    def _init():
        s_ref[...] = jnp.zeros((8, H), jnp.float32)

    s_ref[0:1, :] += jnp.sum(z, axis=0, keepdims=True)
    s_ref[1:2, :] += jnp.sum(z * z, axis=0, keepdims=True)


def _b_body(z_ref, s_ref, g_ref, be_ref, p_ref, h_ref, sc_ref):
    z = z_ref[...]
    mu = s_ref[0:1, :] / N
    var = s_ref[1:2, :] / N - mu * mu
    h = jax.nn.relu((z - mu) / jnp.sqrt(var + EPS) * g_ref[...] + be_ref[...])
    h_ref[...] = h
    p = p_ref[...]
    pn = jnp.sqrt(jnp.sum(p * p)) + 1e-16
    sc_ref[...] = jnp.dot(h, p, preferred_element_type=jnp.float32) / pn


def _bn2(z, g, b):
    mu = jnp.mean(z, axis=0, keepdims=True)
    d = z - mu
    var = jnp.mean(d * d, axis=0, keepdims=True)
    return d / jnp.sqrt(var + EPS) * g + b


def _c_body(h_ref, bcol_ref, brow_ref, srow_ref,
            W1_ref, b1_ref, g1_ref, be1_ref, W2_ref, b2_ref, g2_ref, be2_ref,
            W3_ref, b3_ref, out_ref):
    batch_row = brow_ref[...]
    score = srow_ref[...]

    u = jax.lax.bitcast_convert_type(score, jnp.int32)
    key = jnp.where(u >= 0, u, _KEY_NEG_OFF - u)

    seg_ids = jax.lax.broadcasted_iota(jnp.int32, (B, N), 0)
    M = batch_row == seg_ids
    M_f = M.astype(jnp.float32)
    counts = jnp.sum(M_f, axis=1, keepdims=True)
    k_col = jnp.ceil(jnp.float32(RATIO) * counts)

    def _cnt_ge(t_col):
        c = jnp.logical_and(key >= t_col, M)
        return jnp.sum(c.astype(jnp.float32), axis=1, keepdims=True)

    ge0 = _cnt_ge(jnp.zeros((B, 1), jnp.int32)) >= k_col
    lo = jnp.where(ge0, jnp.int32(0), _LO0)
    hi = jnp.where(ge0, _HI0, jnp.int32(0))

    def _bs(_, carry):
        lo_, hi_ = carry
        mid = lo_ + ((hi_ - lo_) >> 1)
        ok = _cnt_ge(mid) >= k_col
        return jnp.where(ok, mid, lo_), jnp.where(ok, hi_, mid)

    lo, hi = jax.lax.fori_loop(0, 31, _bs, (lo, hi))
    v_col = lo

    gt_mat = jnp.logical_and(key > v_col, M)
    gt_node = jnp.any(gt_mat, axis=0, keepdims=True)
    cnt_gt = jnp.sum(gt_mat.astype(jnp.float32), axis=1, keepdims=True)
    need = k_col - cnt_gt

    tie_mat = jnp.logical_and(key == v_col, M)
    tie_node = jnp.any(tie_mat, axis=0, keepdims=True)
    tie_f = tie_node.astype(jnp.float32)
    ps = tie_f
    d = 1
    while d < N:
        ps = ps + jnp.concatenate(
            [jnp.zeros((1, d), jnp.float32), ps[:, :-d]], axis=1)
        d *= 2
    prefix = ps - tie_f
    tie_cnt = jnp.sum(tie_mat.astype(jnp.float32), axis=1, keepdims=True)
    cs = tie_cnt
    d = 1
    while d < B:
        cs = cs + jnp.concatenate(
            [jnp.zeros((d, 1), jnp.float32), cs[:-d, :]], axis=0)
        d *= 2
    off_col = cs - tie_cnt
    off_node = jnp.sum(M_f * off_col, axis=0, keepdims=True)
    need_node = jnp.sum(M_f * need, axis=0, keepdims=True)
    keep_tie = jnp.logical_and(tie_node, (prefix - off_node) < need_node)
    keep = jnp.logical_or(gt_node, keep_tie).astype(jnp.float32)

    w_row = jnp.tanh(score) * keep
    inv_cnt = 1.0 / jnp.maximum(counts, 1.0)
    tk_cnt = jnp.sum(M_f * keep, axis=1, keepdims=True)
    inv_tk = 1.0 / jnp.maximum(tk_cnt, 1.0)

    h = h_ref[...]
    add_pool = jnp.dot(M_f, h, preferred_element_type=jnp.float32, precision=jax.lax.Precision.HIGHEST)
    mean_pool = jnp.dot(M_f * inv_cnt, h, preferred_element_type=jnp.float32, precision=jax.lax.Precision.HIGHEST)
    topk_mean = jnp.dot(M_f * w_row * inv_tk, h,
                        preferred_element_type=jnp.float32, precision=jax.lax.Precision.HIGHEST)

    batch_col = bcol_ref[...]
    sames = []
    d = 1
    while d < N:
        sh_b = jnp.concatenate(
            [jnp.full((d, 1), -1, jnp.int32), batch_col[:-d, :]], axis=0)
        sames.append(batch_col == sh_b)
        d *= 2
    nxt_row = jnp.concatenate(
        [batch_row[:, 1:], jnp.full((1, 1), -1, jnp.int32)], axis=1)
    is_end = batch_row != nxt_row
    E_f = M_f * is_end.astype(jnp.float32)
    chunks = []
    for c0 in range(0, H, 128):
        cur = h[:, c0:c0 + 128]
        d = 1
        j = 0
        while d < N:
            sh_h = jnp.concatenate(
                [jnp.zeros((d, 128), jnp.float32), cur[:-d, :]], axis=0)
            cur = jnp.maximum(cur, jnp.where(sames[j], sh_h, 0.0))
            d *= 2
            j += 1
        chunks.append(jnp.dot(E_f, cur, preferred_element_type=jnp.float32, precision=jax.lax.Precision.HIGHEST))
    max_pool = jnp.concatenate(chunks, axis=1)

    gfeat = jnp.concatenate([max_pool, mean_pool, add_pool, topk_mean],
                            axis=1)

    dn = (((1,), (1,)), ((), ()))
    h1 = jax.nn.relu(_bn2(
        jax.lax.dot_general(gfeat, W1_ref[...], dn,
                            preferred_element_type=jnp.float32, precision=None)
        + b1_ref[...], g1_ref[...], be1_ref[...]))
    h2 = jax.nn.relu(_bn2(
        jax.lax.dot_general(h1, W2_ref[...], dn,
                            preferred_element_type=jnp.float32, precision=None)
        + b2_ref[...], g2_ref[...], be2_ref[...]))
    out_ref[...] = jax.lax.dot_general(
        h2, W3_ref[...], dn, preferred_element_type=jnp.float32, precision=None) + b3_ref[...]


def _a_call():
    return dict(
        grid=(NSTEP,),
        in_specs=[
            pl.BlockSpec((BLK, D), lambda i: (i, 0)),
            pl.BlockSpec((H, D), lambda i: (0, 0)),
            pl.BlockSpec((1, H), lambda i: (0, 0)),
        ],
        out_specs=[
            pl.BlockSpec((BLK, H), lambda i: (i, 0)),
            pl.BlockSpec((8, H), lambda i: (0, 0)),
        ],
        out_shape=[
            jax.ShapeDtypeStruct((N, H), jnp.float32),
            jax.ShapeDtypeStruct((8, H), jnp.float32),
        ],
    )


def _b_call():
    return dict(
        grid=(NSTEP,),
        in_specs=[
            pl.BlockSpec((BLK, H), lambda i: (i, 0)),
            pl.BlockSpec((8, H), lambda i: (0, 0)),
            pl.BlockSpec((1, H), lambda i: (0, 0)),
            pl.BlockSpec((1, H), lambda i: (0, 0)),
            pl.BlockSpec((H, 1), lambda i: (0, 0)),
        ],
        out_specs=[
            pl.BlockSpec((BLK, H), lambda i: (i, 0)),
            pl.BlockSpec((BLK, 1), lambda i: (i, 0)),
        ],
        out_shape=[
            jax.ShapeDtypeStruct((N, H), jnp.float32),
            jax.ShapeDtypeStruct((N, 1), jnp.float32),
        ],
    )


def _c_call():
    return dict(
        out_shape=jax.ShapeDtypeStruct((B, OUT), jnp.float32),
    )


@jax.jit
def kernel(x, edge_index, batch, W_in, b_in, g_in, be_in, p_topk,
           W1, b1, g1, be1, W2, b2, g2, be2, W3, b3):
    del edge_index
    r = lambda a: a.reshape(1, -1)
    z, stats = pl.pallas_call(_a_body, **_a_call())(x, W_in, r(b_in))
    h, score = pl.pallas_call(_b_body, **_b_call())(
        z, stats, r(g_in), r(be_in), p_topk.reshape(H, 1))
    bcol = batch.astype(jnp.int32).reshape(N, 1)
    return pl.pallas_call(_c_body, **_c_call())(
        h, bcol, bcol.reshape(1, N), score.reshape(1, N),
        W1, r(b1), r(g1), r(be1), W2, r(b2), r(g2), r(be2), W3, r(b3))
